# Initial kernel scaffold; baseline (speedup 1.0000x reference)
import jax
import jax.numpy as jnp
from jax.experimental import pallas as pl


def kernel(x, y):
    raise NotImplementedError("write your pallas kernel here")


# speedup vs baseline: 25.2946x; 25.2946x over previous
import jax
import jax.numpy as jnp
from jax.experimental import pallas as pl
from jax.experimental.pallas import tpu as pltpu

_S = 3.0 ** -0.5
_ROWS = 1000


def _tc_body(x_ref, y_ref, o_ref):
    x = x_ref[...]
    y = y_ref[...]
    x0 = x[:, 0:128]
    x1 = x[:, 128:256]
    x2 = x[:, 256:384]
    x3 = x[:, 384:512]
    y0 = y[:, 0:1]
    y1 = y[:, 1:2]
    y2 = y[:, 2:3]
    y3 = y[:, 3:4]
    o_ref[:, 0:128] = x0 * y0
    o_ref[:, 128:256] = x0 * y1
    o_ref[:, 256:384] = x0 * y2
    o_ref[:, 384:512] = x0 * y3
    o_ref[:, 512:640] = x1 * y0
    o_ref[:, 640:768] = x2 * y0
    o_ref[:, 768:896] = x3 * y0
    o_ref[:, 896:1024] = (x1 * y1 + x2 * y2 + x3 * y3) * _S


def kernel(x, y):
    n = x.shape[0]
    xf = x.reshape(n, 512)
    yf = y.reshape(n, 4)
    grid = (n // _ROWS,)
    out = pl.pallas_call(
        _tc_body,
        grid=grid,
        in_specs=[
            pl.BlockSpec((_ROWS, 512), lambda i: (i, 0)),
            pl.BlockSpec((_ROWS, 4), lambda i: (i, 0)),
        ],
        out_specs=pl.BlockSpec((_ROWS, 1024), lambda i: (i, 0)),
        out_shape=jax.ShapeDtypeStruct((n, 1024), jnp.float32),
    )(xf, yf)
    return out.reshape(n, 8, 128)
